# baseline (speedup 1.0000x reference)
import functools

import jax
import jax.numpy as jnp
from jax import lax
from jax.experimental import pallas as pl
from jax.experimental.pallas import tpu as pltpu
from jax.experimental.pallas import tpu_sc as plsc

BATCH = 16384
DIMS = (16, 24, 32)
CH = 128


@functools.lru_cache(maxsize=None)
def _make_kernel():
    info = plsc.get_sparse_core_info()
    nc, ns = info.num_cores, info.num_subcores
    nw = nc * ns
    b_per_w = BATCH // nw
    n_ch = b_per_w // CH

    mesh = plsc.VectorSubcoreMesh(core_axis_name="c", subcore_axis_name="s")

    @functools.partial(
        pl.kernel,
        mesh=mesh,
        out_type=[
            jax.ShapeDtypeStruct((BATCH, d), jnp.float32) for d in DIMS
        ],
        scratch_types=[
            pltpu.VMEM((3, n_ch, CH), jnp.int32),
            pltpu.VMEM((b_per_w, DIMS[0]), jnp.float32),
            pltpu.VMEM((b_per_w, DIMS[1]), jnp.float32),
            pltpu.VMEM((b_per_w, DIMS[2]), jnp.float32),
            pltpu.SemaphoreType.DMA,
        ],
        compiler_params=pltpu.CompilerParams(use_tc_tiling_on_sc=False),
    )
    def grid_gather(idx_hbm, e0, e1, e2, o0, o1, o2, idx_v, r0, r1, r2, sem):
        wid = lax.axis_index("s") * nc + lax.axis_index("c")
        base = wid * b_per_w
        pltpu.sync_copy(idx_hbm.at[wid], idx_v)
        copies = []
        for j, (tbl, rbuf) in enumerate(((e0, r0), (e1, r1), (e2, r2))):
            for c in range(n_ch):
                copies.append(
                    pltpu.async_copy(
                        tbl.at[idx_v.at[j, c]],
                        rbuf.at[pl.ds(c * CH, CH)],
                        sem,
                    )
                )
        for cp in copies:
            cp.wait()
        pltpu.sync_copy(r0, o0.at[pl.ds(base, b_per_w)])
        pltpu.sync_copy(r1, o1.at[pl.ds(base, b_per_w)])
        pltpu.sync_copy(r2, o2.at[pl.ds(base, b_per_w)])

    nw_const = nw
    n_ch_const = n_ch

    def run(grid_idx, e0, e1, e2):
        idx = jnp.transpose(grid_idx.astype(jnp.int32), (1, 0))
        idx = idx.reshape(3, nw_const, n_ch_const, CH).transpose(1, 0, 2, 3)
        o0, o1, o2 = grid_gather(idx, e0, e1, e2)
        return jnp.concatenate([o0, o1, o2], axis=1)

    return run


def kernel(grid_idx, E0, E1, E2):
    return _make_kernel()(grid_idx, E0, E1, E2)


# speedup vs baseline: 1.0356x; 1.0356x over previous
import functools

import jax
import jax.numpy as jnp
from jax import lax
from jax.experimental import pallas as pl
from jax.experimental.pallas import tpu as pltpu
from jax.experimental.pallas import tpu_sc as plsc

BATCH = 16384
DIMS = (16, 24, 32)
OUT_D = 72
CH = 128
L = 16


@functools.lru_cache(maxsize=None)
def _make_kernel():
    info = plsc.get_sparse_core_info()
    nc, ns = info.num_cores, info.num_subcores
    nw = nc * ns
    b_per_w = BATCH // nw
    n_ch = b_per_w // CH
    g_per_ch = CH // L

    mesh = plsc.VectorSubcoreMesh(core_axis_name="c", subcore_axis_name="s")

    @functools.partial(
        pl.kernel,
        mesh=mesh,
        out_type=jax.ShapeDtypeStruct((BATCH, OUT_D), jnp.float32),
        scratch_types=[
            pltpu.VMEM((b_per_w, 3), jnp.int32),
            pltpu.VMEM((3, n_ch, CH), jnp.int32),
            pltpu.VMEM((b_per_w, DIMS[0]), jnp.float32),
            pltpu.VMEM((b_per_w, DIMS[1]), jnp.float32),
            pltpu.VMEM((b_per_w, DIMS[2]), jnp.float32),
            pltpu.SemaphoreType.DMA,
        ],
        compiler_params=pltpu.CompilerParams(
            use_tc_tiling_on_sc=False, needs_layout_passes=False
        ),
    )
    def grid_gather(gidx, e0, e1, e2, out, raw_v, idx_v, r0, r1, r2, sem):
        wid = lax.axis_index("s") * nc + lax.axis_index("c")
        base = wid * b_per_w
        pltpu.sync_copy(gidx.at[pl.ds(base, b_per_w)], raw_v)
        lane = lax.broadcasted_iota(jnp.int32, (L,), 0)
        for c in range(n_ch):
            for g in range(g_per_ch):
                rows = lane + (c * CH + g * L)
                for j in range(3):
                    cols = jnp.full((L,), j, jnp.int32)
                    vals = plsc.load_gather(raw_v, [rows, cols])
                    idx_v[j, c, pl.ds(g * L, L)] = vals
        copies = []
        for j, (tbl, rbuf) in enumerate(((e0, r0), (e1, r1), (e2, r2))):
            for c in range(n_ch):
                copies.append(
                    pltpu.async_copy(
                        tbl.at[idx_v.at[j, c]],
                        rbuf.at[pl.ds(c * CH, CH)],
                        sem,
                    )
                )
        for cp in copies:
            cp.wait()
        col = 0
        for rbuf, d in ((r0, DIMS[0]), (r1, DIMS[1]), (r2, DIMS[2])):
            pltpu.sync_copy(
                rbuf, out.at[pl.ds(base, b_per_w), pl.ds(col, d)]
            )
            col += d

    def run(grid_idx, e0, e1, e2):
        return grid_gather(grid_idx.astype(jnp.int32), e0, e1, e2)

    return run


def kernel(grid_idx, E0, E1, E2):
    return _make_kernel()(grid_idx, E0, E1, E2)
